# Initial kernel scaffold; baseline (speedup 1.0000x reference)
import jax
import jax.numpy as jnp
from jax.experimental import pallas as pl


def kernel(hidden_states, router_weight, router_bias, gate_up_proj, gate_up_proj_bias, down_proj, down_proj_bias):
    raise NotImplementedError("write your pallas kernel here")


# speedup vs baseline: 4.3310x; 4.3310x over previous
import functools

import jax
import jax.numpy as jnp
from jax.experimental import pallas as pl
from jax.experimental.pallas import tpu as pltpu

B, S, H = 1, 2048, 768
E, K, INTER = 8, 2, 768
ALPHA, LIMIT = 1.702, 7.0

TT = 512


def _moe_kernel(x_ref, rw_ref, rb_ref, gup_ref, gub_ref, dp_ref, db_ref,
                out_ref, scores_ref):
    e = pl.program_id(1)
    x = x_ref[...]

    @pl.when(e == 0)
    def _router():
        logits = jax.lax.dot_general(
            x.astype(jnp.bfloat16), rw_ref[...].astype(jnp.bfloat16),
            (((1,), (1,)), ((), ())),
            preferred_element_type=jnp.float32)
        logits = logits + rb_ref[...]
        idx = jax.lax.broadcasted_iota(jnp.int32, (TT, E), 1)
        m1 = jnp.max(logits, axis=1, keepdims=True)
        i1 = jnp.min(jnp.where(logits == m1, idx, E), axis=1, keepdims=True)
        sel1 = idx == i1
        masked = jnp.where(sel1, -jnp.inf, logits)
        m2 = jnp.max(masked, axis=1, keepdims=True)
        i2 = jnp.min(jnp.where(masked == m2, idx, E), axis=1, keepdims=True)
        sel2 = idx == i2
        e2 = jnp.exp(m2 - m1)
        denom = 1.0 + e2
        p1 = 1.0 / denom
        p2 = e2 / denom
        scores_ref[...] = jnp.where(sel1, p1, 0.0) + jnp.where(sel2, p2, 0.0)

    scores = scores_ref[...]
    idx = jax.lax.broadcasted_iota(jnp.int32, (TT, E), 1)
    w_e = jnp.sum(jnp.where(idx == e, scores, 0.0), axis=1, keepdims=True)

    xb = x.astype(jnp.bfloat16)
    gu = jnp.dot(xb, gup_ref[0], preferred_element_type=jnp.float32)
    gu = gu + gub_ref[0]
    gate = jnp.minimum(gu[:, :INTER], LIMIT)
    up = jnp.clip(gu[:, INTER:], -LIMIT, LIMIT)
    glu = gate * jax.nn.sigmoid(gate * ALPHA)
    act = (up + 1.0) * glu
    dd = jnp.dot(act.astype(jnp.bfloat16), dp_ref[0],
                 preferred_element_type=jnp.float32)
    dd = dd + db_ref[0]
    contrib = dd * w_e

    @pl.when(e == 0)
    def _init():
        out_ref[...] = contrib

    @pl.when(e != 0)
    def _acc():
        out_ref[...] += contrib


@functools.partial(jax.jit, static_argnames=())
def kernel(hidden_states, router_weight, router_bias, gate_up_proj,
           gate_up_proj_bias, down_proj, down_proj_bias):
    b, s, h = hidden_states.shape
    T = b * s
    x = hidden_states.reshape(T, h)
    gup = jnp.concatenate(
        [gate_up_proj[..., 0::2], gate_up_proj[..., 1::2]], axis=-1)
    gup = gup.astype(jnp.bfloat16)
    gub = jnp.concatenate(
        [gate_up_proj_bias[..., 0::2], gate_up_proj_bias[..., 1::2]], axis=-1)
    gub = gub.reshape(E, 1, 2 * INTER)
    dp = down_proj.astype(jnp.bfloat16)
    db = down_proj_bias.reshape(E, 1, H)
    rb = router_bias.reshape(1, E)

    n_t = T // TT
    grid = (n_t, E)
    out, scores = pl.pallas_call(
        _moe_kernel,
        grid=grid,
        in_specs=[
            pl.BlockSpec((TT, H), lambda t, e: (t, 0)),
            pl.BlockSpec((E, H), lambda t, e: (0, 0)),
            pl.BlockSpec((1, E), lambda t, e: (0, 0)),
            pl.BlockSpec((1, H, 2 * INTER), lambda t, e: (e, 0, 0)),
            pl.BlockSpec((1, 1, 2 * INTER), lambda t, e: (e, 0, 0)),
            pl.BlockSpec((1, INTER, H), lambda t, e: (e, 0, 0)),
            pl.BlockSpec((1, 1, H), lambda t, e: (e, 0, 0)),
        ],
        out_specs=[
            pl.BlockSpec((TT, H), lambda t, e: (t, 0)),
            pl.BlockSpec((TT, E), lambda t, e: (t, 0)),
        ],
        out_shape=[
            jax.ShapeDtypeStruct((T, H), jnp.float32),
            jax.ShapeDtypeStruct((T, E), jnp.float32),
        ],
        compiler_params=pltpu.CompilerParams(
            dimension_semantics=("parallel", "arbitrary"),
        ),
    )(x, router_weight, rb, gup, gub, dp, db)

    return out.reshape(b, s, h), scores


# speedup vs baseline: 10.7020x; 2.4710x over previous
import functools

import jax
import jax.numpy as jnp
from jax.experimental import pallas as pl
from jax.experimental.pallas import tpu as pltpu

B, S, H = 1, 2048, 768
E, K, INTER = 8, 2, 768
ALPHA, LIMIT = 1.702, 7.0

TT = 512


def _moe_kernel(x_ref, rw_ref, rb_ref, gup_ref, gub_ref, dp_ref, db_ref,
                out_ref, scores_ref):
    e = pl.program_id(1)
    x = x_ref[...]

    @pl.when(e == 0)
    def _router():
        logits = jax.lax.dot_general(
            x.astype(jnp.bfloat16), rw_ref[...].astype(jnp.bfloat16),
            (((1,), (1,)), ((), ())),
            preferred_element_type=jnp.float32)
        logits = logits + rb_ref[...]
        idx = jax.lax.broadcasted_iota(jnp.int32, (TT, E), 1)
        m1 = jnp.max(logits, axis=1, keepdims=True)
        i1 = jnp.min(jnp.where(logits == m1, idx, E), axis=1, keepdims=True)
        sel1 = idx == i1
        masked = jnp.where(sel1, -jnp.inf, logits)
        m2 = jnp.max(masked, axis=1, keepdims=True)
        i2 = jnp.min(jnp.where(masked == m2, idx, E), axis=1, keepdims=True)
        sel2 = idx == i2
        e2 = jnp.exp(m2 - m1)
        denom = 1.0 + e2
        p1 = 1.0 / denom
        p2 = e2 / denom
        scores_ref[...] = jnp.where(sel1, p1, 0.0) + jnp.where(sel2, p2, 0.0)

    scores = scores_ref[...]
    idx = jax.lax.broadcasted_iota(jnp.int32, (TT, E), 1)
    w_e = jnp.sum(jnp.where(idx == e, scores, 0.0), axis=1, keepdims=True)

    xb = x.astype(jnp.bfloat16)
    gu = jnp.dot(xb, gup_ref[0],
                 preferred_element_type=jnp.float32) + gub_ref[0]
    gu_up = pltpu.roll(gu, 2 * INTER - 1, 1)
    gate = jnp.minimum(gu, LIMIT)
    up = jnp.clip(gu_up, -LIMIT, LIMIT)
    glu = gate * jax.nn.sigmoid(gate * ALPHA)
    act = (up + 1.0) * glu
    lane = jax.lax.broadcasted_iota(jnp.int32, (TT, 2 * INTER), 1)
    act = jnp.where(lane % 2 == 0, act, 0.0)
    dd = jnp.dot(act.astype(jnp.bfloat16), dp_ref[0],
                 preferred_element_type=jnp.float32)
    dd = dd + db_ref[0]
    contrib = dd * w_e

    @pl.when(e == 0)
    def _init():
        out_ref[...] = contrib

    @pl.when(e != 0)
    def _acc():
        out_ref[...] += contrib


@functools.partial(jax.jit, static_argnames=())
def kernel(hidden_states, router_weight, router_bias, gate_up_proj,
           gate_up_proj_bias, down_proj, down_proj_bias):
    b, s, h = hidden_states.shape
    T = b * s
    x = hidden_states.reshape(T, h)
    gup = gate_up_proj.astype(jnp.bfloat16)
    gub = gate_up_proj_bias.reshape(E, 1, 2 * INTER)
    dpb = down_proj.astype(jnp.bfloat16)
    dp = jnp.concatenate(
        [dpb[:, :, None, :], jnp.zeros_like(dpb)[:, :, None, :]],
        axis=2).reshape(E, 2 * INTER, H)
    db = down_proj_bias.reshape(E, 1, H)
    rb = router_bias.reshape(1, E)

    n_t = T // TT
    grid = (n_t, E)
    out, scores = pl.pallas_call(
        _moe_kernel,
        grid=grid,
        in_specs=[
            pl.BlockSpec((TT, H), lambda t, e: (t, 0)),
            pl.BlockSpec((E, H), lambda t, e: (0, 0)),
            pl.BlockSpec((1, E), lambda t, e: (0, 0)),
            pl.BlockSpec((1, H, 2 * INTER), lambda t, e: (e, 0, 0)),
            pl.BlockSpec((1, 1, 2 * INTER), lambda t, e: (e, 0, 0)),
            pl.BlockSpec((1, 2 * INTER, H), lambda t, e: (e, 0, 0)),
            pl.BlockSpec((1, 1, H), lambda t, e: (e, 0, 0)),
        ],
        out_specs=[
            pl.BlockSpec((TT, H), lambda t, e: (t, 0)),
            pl.BlockSpec((TT, E), lambda t, e: (t, 0)),
        ],
        out_shape=[
            jax.ShapeDtypeStruct((T, H), jnp.float32),
            jax.ShapeDtypeStruct((T, E), jnp.float32),
        ],
        compiler_params=pltpu.CompilerParams(
            dimension_semantics=("parallel", "arbitrary"),
        ),
    )(x, router_weight, rb, gup, gub, dp, db)

    return out.reshape(b, s, h), scores


# speedup vs baseline: 21.4183x; 2.0013x over previous
import functools

import jax
import jax.numpy as jnp
from jax.experimental import pallas as pl
from jax.experimental.pallas import tpu as pltpu

B, S, H = 1, 2048, 768
E, K, INTER = 8, 2, 768
ALPHA, LIMIT = 1.702, 7.0

TT = 1024


def _moe_kernel(x_ref, rw_ref, rb_ref, gup_ref, gub_ref, dp_ref, db_ref,
                out_ref, scores_ref):
    e = pl.program_id(1)
    x = x_ref[...]

    @pl.when(e == 0)
    def _router():
        logits = jax.lax.dot_general(
            x.astype(jnp.bfloat16), rw_ref[...].astype(jnp.bfloat16),
            (((1,), (1,)), ((), ())),
            preferred_element_type=jnp.float32)
        logits = logits + rb_ref[...]
        idx = jax.lax.broadcasted_iota(jnp.int32, (TT, E), 1)
        m1 = jnp.max(logits, axis=1, keepdims=True)
        i1 = jnp.min(jnp.where(logits == m1, idx, E), axis=1, keepdims=True)
        sel1 = idx == i1
        masked = jnp.where(sel1, -jnp.inf, logits)
        m2 = jnp.max(masked, axis=1, keepdims=True)
        i2 = jnp.min(jnp.where(masked == m2, idx, E), axis=1, keepdims=True)
        sel2 = idx == i2
        e2 = jnp.exp(m2 - m1)
        denom = 1.0 + e2
        p1 = 1.0 / denom
        p2 = e2 / denom
        scores_ref[...] = jnp.where(sel1, p1, 0.0) + jnp.where(sel2, p2, 0.0)

    scores = scores_ref[...]
    idx = jax.lax.broadcasted_iota(jnp.int32, (TT, E), 1)
    w_e = jnp.sum(jnp.where(idx == e, scores, 0.0), axis=1, keepdims=True)

    xb = x.astype(jnp.bfloat16)
    gu = jnp.dot(xb, gup_ref[0].astype(jnp.bfloat16),
                 preferred_element_type=jnp.float32) + gub_ref[0]
    gu_up = pltpu.roll(gu, 2 * INTER - 1, 1)
    gate = jnp.minimum(gu, LIMIT)
    up = jnp.clip(gu_up, -LIMIT, LIMIT)
    glu = gate * jax.nn.sigmoid(gate * ALPHA)
    act = (up + 1.0) * glu
    lane = jax.lax.broadcasted_iota(jnp.int32, (TT, 2 * INTER), 1)
    act = jnp.where(lane % 2 == 0, act, 0.0)
    dpv = dp_ref[0].astype(jnp.bfloat16)
    dp2 = jnp.concatenate(
        [dpv[:, None, :], jnp.zeros_like(dpv)[:, None, :]],
        axis=1).reshape(2 * INTER, H)
    dd = jnp.dot(act.astype(jnp.bfloat16), dp2,
                 preferred_element_type=jnp.float32)
    dd = dd + db_ref[0]
    contrib = dd * w_e

    @pl.when(e == 0)
    def _init():
        out_ref[...] = contrib

    @pl.when(e != 0)
    def _acc():
        out_ref[...] += contrib


@functools.partial(jax.jit, static_argnames=())
def kernel(hidden_states, router_weight, router_bias, gate_up_proj,
           gate_up_proj_bias, down_proj, down_proj_bias):
    b, s, h = hidden_states.shape
    T = b * s
    x = hidden_states.reshape(T, h)
    gup = gate_up_proj
    gub = gate_up_proj_bias.reshape(E, 1, 2 * INTER)
    dp = down_proj
    db = down_proj_bias.reshape(E, 1, H)
    rb = router_bias.reshape(1, E)

    n_t = T // TT
    grid = (n_t, E)
    out, scores = pl.pallas_call(
        _moe_kernel,
        grid=grid,
        in_specs=[
            pl.BlockSpec((TT, H), lambda t, e: (t, 0)),
            pl.BlockSpec((E, H), lambda t, e: (0, 0)),
            pl.BlockSpec((1, E), lambda t, e: (0, 0)),
            pl.BlockSpec((1, H, 2 * INTER), lambda t, e: (e, 0, 0)),
            pl.BlockSpec((1, 1, 2 * INTER), lambda t, e: (e, 0, 0)),
            pl.BlockSpec((1, INTER, H), lambda t, e: (e, 0, 0)),
            pl.BlockSpec((1, 1, H), lambda t, e: (e, 0, 0)),
        ],
        out_specs=[
            pl.BlockSpec((TT, H), lambda t, e: (t, 0)),
            pl.BlockSpec((TT, E), lambda t, e: (t, 0)),
        ],
        out_shape=[
            jax.ShapeDtypeStruct((T, H), jnp.float32),
            jax.ShapeDtypeStruct((T, E), jnp.float32),
        ],
        compiler_params=pltpu.CompilerParams(
            dimension_semantics=("parallel", "arbitrary"),
        ),
    )(x, router_weight, rb, gup, gub, dp, db)

    return out.reshape(b, s, h), scores
